# baseline (speedup 1.0000x reference)
import functools

import jax
import jax.numpy as jnp
from jax import lax
from jax.experimental import pallas as pl
from jax.experimental.pallas import tpu as pltpu
from jax.experimental.pallas import tpu_sc as plsc

_B, _L, _V, _D = 16384, 50, 1000000, 64
_M, _MD, _O = 100, 64, 128
_PW = 2 * _D

_NC, _NS = 2, 16
_NW = _NC * _NS
_RPW = _B // _NW
_RC = 2
_LP = 56
_CROWS = _RC * _LP
_NCHUNK = _RPW // _RC


def _gather_max_body(idxp_hbm, qoff_hbm, table_hbm, out_hbm,
                     idxp_v, qoff_v, rows0, rows1, out_v, sem0, sem1):
    wid = lax.axis_index("s") * _NC + lax.axis_index("c")
    rule0 = wid * _RPW

    pltpu.sync_copy(idxp_hbm.at[pl.ds(rule0, _RPW), pl.ds(0, _LP)], idxp_v)
    pltpu.sync_copy(qoff_hbm.at[pl.ds(rule0, _RPW), pl.ds(0, _LP)], qoff_v)

    def fire(c, rows, sem):
        for r in range(_RC):
            pltpu.async_copy(
                table_hbm.at[idxp_v.at[c * _RC + r]],
                rows.at[pl.ds(r * _LP, _LP)],
                sem,
            )

    def compute(c, rows):
        def rule_body(r, carry):
            row = c * _RC + r
            base = r * _LP
            qs = [qoff_v[row, pl.ds(g * 16, 16)] for g in range(3)]
            qs.append(qoff_v[row, pl.ds(40, 16)])

            def off_for(l):
                g, lane = (l // 16, l % 16) if l < 48 else (3, l - 40)
                return pl.multiple_of(qs[g][lane], 64)

            off = off_for(0)
            accs = [rows[base, pl.ds(off + cg * 16, 16)] for cg in range(4)]
            for l in range(1, _L):
                off = off_for(l)
                for cg in range(4):
                    accs[cg] = jnp.maximum(
                        accs[cg], rows[base + l, pl.ds(off + cg * 16, 16)]
                    )
            for cg in range(4):
                out_v[row, pl.ds(cg * 16, 16)] = accs[cg]
            return carry

        lax.fori_loop(0, _RC, rule_body, 0, unroll=False)

    def half(c, rows_cur, sem_cur, rows_nxt, sem_nxt):
        @pl.when(c + 1 < _NCHUNK)
        def _():
            fire(c + 1, rows_nxt, sem_nxt)

        pltpu.make_async_copy(
            table_hbm.at[pl.ds(0, _CROWS)], rows_cur, sem_cur
        ).wait()
        compute(c, rows_cur)

    fire(0, rows0, sem0)

    def pair(k, carry):
        half(2 * k, rows0, sem0, rows1, sem1)
        half(2 * k + 1, rows1, sem1, rows0, sem0)
        return carry

    lax.fori_loop(0, _NCHUNK // 2, pair, 0, unroll=False)

    pltpu.sync_copy(out_v, out_hbm.at[pl.ds(rule0, _RPW)])


@jax.jit
def _gather_max(idxp, qoff, tabv):
    mesh = plsc.VectorSubcoreMesh(core_axis_name="c", subcore_axis_name="s")
    return pl.kernel(
        _gather_max_body,
        out_type=jax.ShapeDtypeStruct((_B, _D), jnp.float32),
        mesh=mesh,
        scratch_types=[
            pltpu.VMEM((_RPW, _LP), jnp.int32),
            pltpu.VMEM((_RPW, _LP), jnp.int32),
            pltpu.VMEM((_CROWS, _PW), jnp.float32),
            pltpu.VMEM((_CROWS, _PW), jnp.float32),
            pltpu.VMEM((_RPW, _D), jnp.float32),
            pltpu.SemaphoreType.DMA,
            pltpu.SemaphoreType.DMA,
        ],
        compiler_params=pltpu.CompilerParams(use_tc_tiling_on_sc=False),
    )(idxp, qoff, tabv)


def _mlp_body(pred_ref, motif_ref, w1_ref, b1_ref, w2_ref, b2_ref,
              w3a_ref, w3b_ref, b3_ref, out_ref):
    h = jnp.dot(motif_ref[...], w1_ref[...], preferred_element_type=jnp.float32)
    h = jnp.maximum(h + b1_ref[...], 0.0)
    m = jnp.dot(h, w2_ref[...], preferred_element_type=jnp.float32)
    m = jnp.maximum(m + b2_ref[...], 0.0)
    o = jnp.dot(pred_ref[...], w3a_ref[...], preferred_element_type=jnp.float32)
    o = o + jnp.dot(m, w3b_ref[...], preferred_element_type=jnp.float32)
    out_ref[...] = jnp.maximum(o + b3_ref[...], 0.0)


@jax.jit
def _mlp(pred, motif, W1, b1, W2, b2, W3a, W3b, b3):
    bb = 2048
    grid = (_B // bb,)
    rep = lambda shape: pl.BlockSpec(shape, lambda i: (0,) * len(shape))
    return pl.pallas_call(
        _mlp_body,
        grid=grid,
        in_specs=[
            pl.BlockSpec((bb, _D), lambda i: (i, 0)),
            pl.BlockSpec((bb, _M), lambda i: (i, 0)),
            rep((_M, _MD)), rep((1, _MD)),
            rep((_MD, _MD)), rep((1, _MD)),
            rep((_D, _O)), rep((_MD, _O)), rep((1, _O)),
        ],
        out_specs=pl.BlockSpec((bb, _O), lambda i: (i, 0)),
        out_shape=jax.ShapeDtypeStruct((_B, _O), jnp.float32),
    )(pred, motif, W1, b1, W2, b2, W3a, W3b, b3)


def kernel(predicate_indices_list, motif_counts_batch, table, W1, b1, W2, b2, W3, b3):
    idx_pad = jnp.pad(predicate_indices_list, ((0, 0), (0, 128 - _L)),
                      mode="edge")
    half_v = _V // 2
    idxp = jnp.where(idx_pad < half_v, idx_pad, idx_pad - half_v)
    qoff = (idx_pad >= half_v).astype(jnp.int32) << 6
    tabv = jnp.concatenate([table[:half_v], table[half_v:]], axis=1)
    pred = _gather_max(idxp, qoff, tabv)
    return _mlp(
        pred, motif_counts_batch,
        W1, b1.reshape(1, _MD),
        W2, b2.reshape(1, _MD),
        W3[:_D], W3[_D:], b3.reshape(1, _O),
    )


# speedup vs baseline: 1.4088x; 1.4088x over previous
import functools

import jax
import jax.numpy as jnp
from jax import lax
from jax.experimental import pallas as pl
from jax.experimental.pallas import tpu as pltpu
from jax.experimental.pallas import tpu_sc as plsc

_B, _L, _V, _D = 16384, 50, 1000000, 64
_M, _MD, _O = 100, 64, 128

_NC, _NS = 2, 16
_NW = _NC * _NS
_RPW = _B // _NW
_RC = 8
_CROWS = _RC * _L
_NCHUNK = _RPW // _RC


def _gather_max_body(idx_hbm, table_hbm, out_hbm,
                     idx_v, rows0, rows1, out_v, sem0, sem1):
    wid = lax.axis_index("s") * _NC + lax.axis_index("c")
    rule0 = wid * _RPW

    pltpu.sync_copy(idx_hbm.at[pl.ds(rule0, _RPW)], idx_v)

    def fire(c, rows, sem):
        for r in range(_RC):
            pltpu.async_copy(
                table_hbm.at[idx_v.at[c * _RC + r]],
                rows.at[pl.ds(r * _L, _L)],
                sem,
            )

    def compute(c, rows):
        def rule_body(r, carry):
            base = r * _L
            accs = [rows[base, pl.ds(cg * 16, 16)] for cg in range(_D // 16)]
            for l in range(1, _L):
                for cg in range(_D // 16):
                    accs[cg] = jnp.maximum(
                        accs[cg], rows[base + l, pl.ds(cg * 16, 16)]
                    )
            for cg in range(_D // 16):
                out_v[c * _RC + r, pl.ds(cg * 16, 16)] = accs[cg]
            return carry

        lax.fori_loop(0, _RC, rule_body, 0, unroll=False)

    def half(c, rows_cur, sem_cur, rows_nxt, sem_nxt):
        @pl.when(c + 1 < _NCHUNK)
        def _():
            fire(c + 1, rows_nxt, sem_nxt)

        pltpu.make_async_copy(
            table_hbm.at[pl.ds(0, _CROWS)], rows_cur, sem_cur
        ).wait()
        compute(c, rows_cur)

    fire(0, rows0, sem0)

    def pair(k, carry):
        half(2 * k, rows0, sem0, rows1, sem1)
        half(2 * k + 1, rows1, sem1, rows0, sem0)
        return carry

    lax.fori_loop(0, _NCHUNK // 2, pair, 0, unroll=False)

    pltpu.sync_copy(out_v, out_hbm.at[pl.ds(rule0, _RPW)])


@jax.jit
def _gather_max(idx, table):
    mesh = plsc.VectorSubcoreMesh(core_axis_name="c", subcore_axis_name="s")
    return pl.kernel(
        _gather_max_body,
        out_type=jax.ShapeDtypeStruct((_B, _D), jnp.float32),
        mesh=mesh,
        scratch_types=[
            pltpu.VMEM((_RPW, _L), jnp.int32),
            pltpu.VMEM((_CROWS, _D), jnp.float32),
            pltpu.VMEM((_CROWS, _D), jnp.float32),
            pltpu.VMEM((_RPW, _D), jnp.float32),
            pltpu.SemaphoreType.DMA,
            pltpu.SemaphoreType.DMA,
        ],
        compiler_params=pltpu.CompilerParams(use_tc_tiling_on_sc=False),
    )(idx, table)


def _mlp_body(pred_ref, motif_ref, w1_ref, b1_ref, w2_ref, b2_ref,
              w3a_ref, w3b_ref, b3_ref, out_ref):
    h = jnp.dot(motif_ref[...], w1_ref[...], preferred_element_type=jnp.float32)
    h = jnp.maximum(h + b1_ref[...], 0.0)
    m = jnp.dot(h, w2_ref[...], preferred_element_type=jnp.float32)
    m = jnp.maximum(m + b2_ref[...], 0.0)
    o = jnp.dot(pred_ref[...], w3a_ref[...], preferred_element_type=jnp.float32)
    o = o + jnp.dot(m, w3b_ref[...], preferred_element_type=jnp.float32)
    out_ref[...] = jnp.maximum(o + b3_ref[...], 0.0)


@jax.jit
def _mlp(pred, motif, W1, b1, W2, b2, W3a, W3b, b3):
    bb = 2048
    grid = (_B // bb,)
    rep = lambda shape: pl.BlockSpec(shape, lambda i: (0,) * len(shape))
    return pl.pallas_call(
        _mlp_body,
        grid=grid,
        in_specs=[
            pl.BlockSpec((bb, _D), lambda i: (i, 0)),
            pl.BlockSpec((bb, _M), lambda i: (i, 0)),
            rep((_M, _MD)), rep((1, _MD)),
            rep((_MD, _MD)), rep((1, _MD)),
            rep((_D, _O)), rep((_MD, _O)), rep((1, _O)),
        ],
        out_specs=pl.BlockSpec((bb, _O), lambda i: (i, 0)),
        out_shape=jax.ShapeDtypeStruct((_B, _O), jnp.float32),
    )(pred, motif, W1, b1, W2, b2, W3a, W3b, b3)


def kernel(predicate_indices_list, motif_counts_batch, table, W1, b1, W2, b2, W3, b3):
    pred = _gather_max(predicate_indices_list, table)
    return _mlp(
        pred, motif_counts_batch,
        W1, b1.reshape(1, _MD),
        W2, b2.reshape(1, _MD),
        W3[:_D], W3[_D:], b3.reshape(1, _O),
    )
